# baseline (speedup 1.0000x reference)
import functools

import jax
import jax.numpy as jnp
from jax import lax
from jax.experimental import pallas as pl
from jax.experimental.pallas import tpu as pltpu
from jax.experimental.pallas import tpu_sc as plsc

N_NODES = 10000
N_EDGES = 320000
D = 128
ED = 16

NC = 2
NS = 16
NW = NC * NS
WIN = 128

E_PAD = 327680
CH_SIZES = (114688, 106496, 98304, 8192)
K_CH = len(CH_SIZES)
N_AGG = 10112
ROWS_PER_SUB = N_AGG // NS

NBUF = 4
LEAD = 2
EBLK = 1024
NBLK = 400


def _mish(v):
    t = jnp.exp(jnp.minimum(v, 40.0))
    u = t * (t + 2.0)
    return v * (u / (u + 2.0))


def _sc_gather(x, idx_chunk, e_ch):
    mesh = plsc.VectorSubcoreMesh(core_axis_name="c", subcore_axis_name="s")
    GWIN_PER_W = 2 * e_ch // (NW * WIN)

    @functools.partial(
        pl.kernel,
        out_type=jax.ShapeDtypeStruct((2 * e_ch, D), jnp.float32),
        mesh=mesh,
        scratch_types=[
            pltpu.VMEM((GWIN_PER_W, WIN), jnp.int32),
            [pltpu.VMEM((WIN, D), jnp.float32) for _ in range(NBUF)],
            [pltpu.SemaphoreType.DMA for _ in range(NBUF)],
            [pltpu.SemaphoreType.DMA for _ in range(NBUF)],
        ],
    )
    def gather_kernel(x_hbm, i_hbm, g_hbm, ibuf, bufs, gsems, wsems):
        cid = lax.axis_index("c")
        sid = lax.axis_index("s")
        wid = sid * NC + cid
        row0 = wid * GWIN_PER_W * WIN

        pltpu.sync_copy(i_hbm.at[wid], ibuf)

        def gather_desc(g, b):
            return pltpu.make_async_copy(
                x_hbm.at[ibuf.at[g]], bufs[b], gsems[b])

        def wout_desc(g, b):
            return pltpu.make_async_copy(
                bufs[b], g_hbm.at[pl.ds(row0 + g * WIN, WIN)], wsems[b])

        for b in range(NBUF):
            gather_desc(b, b).start()

        @pl.loop(0, GWIN_PER_W, step=NBUF)
        def _(j):
            for b in range(NBUF):
                g = j + b
                gather_desc(g, b).wait()
                wout_desc(g, b).start()
                b2 = (b + LEAD) % NBUF
                g2 = g + LEAD
                gp = g - (NBUF - LEAD)

                @pl.when(jnp.logical_and(gp >= 0, g2 < GWIN_PER_W))
                def _():
                    wout_desc(gp, b2).wait()
                    gather_desc(g2, b2).start()

        for b in range(NBUF):
            wout_desc(GWIN_PER_W - NBUF + b, b).wait()

    return gather_kernel(x, idx_chunk.reshape(NW, GWIN_PER_W, WIN))


def _edge_mlp_body(xs_ref, xd_ref, ea_ref, w1s_ref, w1d_ref, w1e_ref,
                   b1_ref, w2_ref, b2_ref, m_ref):
    bf = jnp.bfloat16
    h = (
        jnp.dot(xs_ref[...].astype(bf), w1s_ref[...], preferred_element_type=jnp.float32)
        + jnp.dot(xd_ref[...].astype(bf), w1d_ref[...], preferred_element_type=jnp.float32)
        + jnp.dot(ea_ref[...], w1e_ref[...], preferred_element_type=jnp.float32)
        + b1_ref[...]
    )
    h = _mish(h)
    h = jnp.dot(h.astype(bf), w2_ref[...],
                preferred_element_type=jnp.float32) + b2_ref[...]
    m_ref[...] = _mish(h)


def _edge_mlp(g, ea_chunk, W1s, W1d, W1e, b1, W2, b2, e_ch):
    nblk = e_ch // EBLK
    return pl.pallas_call(
        _edge_mlp_body,
        grid=(nblk,),
        in_specs=[
            pl.BlockSpec((EBLK, D), lambda i: (i, 0)),
            pl.BlockSpec((EBLK, D), lambda i, _n=nblk: (i + _n, 0)),
            pl.BlockSpec((EBLK, ED), lambda i: (i, 0)),
            pl.BlockSpec((D, D), lambda i: (0, 0)),
            pl.BlockSpec((D, D), lambda i: (0, 0)),
            pl.BlockSpec((ED, D), lambda i: (0, 0)),
            pl.BlockSpec((1, D), lambda i: (0, 0)),
            pl.BlockSpec((D, D), lambda i: (0, 0)),
            pl.BlockSpec((1, D), lambda i: (0, 0)),
        ],
        out_specs=pl.BlockSpec((EBLK, D), lambda i: (i, 0)),
        out_shape=jax.ShapeDtypeStruct((e_ch, D), jnp.float32),
    )(g, g, ea_chunk, W1s, W1d, W1e, b1, W2, b2)


def _sc_scatter_add(m, dst_chunk, e_ch):
    mesh = plsc.VectorSubcoreMesh(core_axis_name="c", subcore_axis_name="s")
    SWIN_PER_W = e_ch // (NW * WIN)

    @functools.partial(
        pl.kernel,
        out_type=jax.ShapeDtypeStruct((NC, N_AGG, D), jnp.float32),
        mesh=mesh,
        scratch_types=[
            pltpu.VMEM((SWIN_PER_W, WIN), jnp.int32),
            [pltpu.VMEM((WIN, D), jnp.float32) for _ in range(2)],
            [pltpu.SemaphoreType.DMA for _ in range(2)],
            pltpu.VMEM_SHARED((N_AGG, D), jnp.float32),
        ],
    )
    def scatter_kernel(m_hbm, di_hbm, out_hbm, idx_v, mbufs, msems, agg_sh):
        cid = lax.axis_index("c")
        sid = lax.axis_index("s")
        wid = sid * NC + cid
        base = wid * SWIN_PER_W * WIN

        def mread_desc(s, u):
            return pltpu.make_async_copy(
                m_hbm.at[pl.ds(base + s * WIN, WIN)], mbufs[u], msems[u])

        @pl.loop(0, WIN)
        def _(r):
            @pl.loop(0, D, step=16)
            def _(q):
                mbufs[0][r, pl.ds(q, 16)] = jnp.zeros((16,), jnp.float32)

        zrows = (WIN, WIN, WIN, WIN, ROWS_PER_SUB - 4 * WIN)
        off = sid * ROWS_PER_SUB
        for i, zr in enumerate(zrows):
            pltpu.sync_copy(
                mbufs[0].at[pl.ds(0, zr)],
                agg_sh.at[pl.ds(off + i * WIN, zr)],
            )

        pltpu.sync_copy(di_hbm.at[wid], idx_v)
        plsc.subcore_barrier()

        mread_desc(0, 0).start()
        mread_desc(1, 1).start()

        @pl.loop(0, SWIN_PER_W, step=2)
        def _(j):
            for u in range(2):
                s = j + u
                mread_desc(s, u).wait()
                pltpu.sync_copy(mbufs[u], agg_sh.at[idx_v.at[s]], add=True)

                @pl.when(s + 2 < SWIN_PER_W)
                def _():
                    mread_desc(s + 2, u).start()

        plsc.subcore_barrier()
        pltpu.sync_copy(
            agg_sh.at[pl.ds(sid * ROWS_PER_SUB, ROWS_PER_SUB)],
            out_hbm.at[cid].at[pl.ds(sid * ROWS_PER_SUB, ROWS_PER_SUB)],
        )

    return scatter_kernel(m, dst_chunk.reshape(NW, SWIN_PER_W, WIN))


def _node_mlp_body(x_ref, *refs):
    p_refs = refs[:K_CH]
    u1x_ref, u1a_ref, c1_ref, u2_ref, c2_ref, g_ref, bt_ref, o_ref = refs[K_CH:]
    x = x_ref[...]
    agg = sum(p[0] + p[1] for p in p_refs)
    u = (
        jnp.dot(x, u1x_ref[...], preferred_element_type=jnp.float32)
        + jnp.dot(agg, u1a_ref[...], preferred_element_type=jnp.float32)
        + c1_ref[...]
    )
    o = jnp.dot(_mish(u), u2_ref[...], preferred_element_type=jnp.float32) + c2_ref[...]
    r = x + o
    mu = jnp.mean(r, axis=-1, keepdims=True)
    var = jnp.mean((r - mu) ** 2, axis=-1, keepdims=True)
    o_ref[...] = (r - mu) * jax.lax.rsqrt(var + 1e-5) * g_ref[...] + bt_ref[...]


def _node_mlp(x, partials, U1, c1, U2, c2, gamma, beta):
    grid = (N_NODES // NBLK,)
    pspec = pl.BlockSpec((NC, NBLK, D), lambda i: (0, i, 0))
    wspec = pl.BlockSpec((D, D), lambda i: (0, 0))
    vspec = pl.BlockSpec((1, D), lambda i: (0, 0))
    return pl.pallas_call(
        _node_mlp_body,
        grid=grid,
        in_specs=[pl.BlockSpec((NBLK, D), lambda i: (i, 0))]
                 + [pspec] * K_CH
                 + [wspec, wspec, vspec, wspec, vspec, vspec, vspec],
        out_specs=pl.BlockSpec((NBLK, D), lambda i: (i, 0)),
        out_shape=jax.ShapeDtypeStruct((N_NODES, D), jnp.float32),
    )(x, *partials, U1[:D], U1[D:], c1.reshape(1, D), U2, c2.reshape(1, D),
      gamma.reshape(1, D), beta.reshape(1, D))


def kernel(x, edge_index, edge_attr, W1, b1, W2, b2, U1, c1, U2, c2, gamma, beta):
    pad = E_PAD - N_EDGES
    src = jnp.concatenate([edge_index[0], jnp.zeros((pad,), jnp.int32)])
    dst = jnp.concatenate([edge_index[1], jnp.full((pad,), N_NODES, jnp.int32)])
    ea = jnp.concatenate([edge_attr, jnp.zeros((pad, ED), jnp.float32)], axis=0)

    bf = jnp.bfloat16
    W1s, W1d = W1[:D].astype(bf), W1[D:2 * D].astype(bf)
    W1e, b1r = W1[2 * D:], b1.reshape(1, D)
    W2b, b2r = W2.astype(bf), b2.reshape(1, D)

    partials = []
    base = 0
    for e_ch in CH_SIZES:
        sl = slice(base, base + e_ch)
        base += e_ch
        idx_k = jnp.concatenate([src[sl], dst[sl]])
        g_k = _sc_gather(x, idx_k, e_ch)
        m_k = _edge_mlp(g_k, ea[sl], W1s, W1d, W1e, b1r, W2b, b2r, e_ch)
        partials.append(_sc_scatter_add(m_k, dst[sl], e_ch))

    return _node_mlp(x, partials, U1, c1, U2, c2, gamma, beta)


# speedup vs baseline: 1.0101x; 1.0101x over previous
import functools

import jax
import jax.numpy as jnp
from jax import lax
from jax.experimental import pallas as pl
from jax.experimental.pallas import tpu as pltpu
from jax.experimental.pallas import tpu_sc as plsc

N_NODES = 10000
N_EDGES = 320000
D = 128
ED = 16

NC = 2
NS = 16
NW = NC * NS
WIN = 128

E_PAD = 327680
CH_SIZES = (106496, 106496, 98304, 16384)
K_CH = len(CH_SIZES)
N_AGG = 10112
ROWS_PER_SUB = N_AGG // NS

NBUF = 4
LEAD = 2
EBLK = 1024
NBLK = 400


def _mish(v):
    t = jnp.exp(jnp.minimum(v, 40.0))
    u = t * (t + 2.0)
    return v * (u / (u + 2.0))


def _sc_gather(x, idx_chunk, e_ch):
    mesh = plsc.VectorSubcoreMesh(core_axis_name="c", subcore_axis_name="s")
    GWIN_PER_W = 2 * e_ch // (NW * WIN)

    @functools.partial(
        pl.kernel,
        out_type=jax.ShapeDtypeStruct((2 * e_ch, D), jnp.float32),
        mesh=mesh,
        scratch_types=[
            pltpu.VMEM((GWIN_PER_W, WIN), jnp.int32),
            [pltpu.VMEM((WIN, D), jnp.float32) for _ in range(NBUF)],
            [pltpu.SemaphoreType.DMA for _ in range(NBUF)],
            [pltpu.SemaphoreType.DMA for _ in range(NBUF)],
        ],
    )
    def gather_kernel(x_hbm, i_hbm, g_hbm, ibuf, bufs, gsems, wsems):
        cid = lax.axis_index("c")
        sid = lax.axis_index("s")
        wid = sid * NC + cid
        row0 = wid * GWIN_PER_W * WIN

        pltpu.sync_copy(i_hbm.at[wid], ibuf)

        def gather_desc(g, b):
            return pltpu.make_async_copy(
                x_hbm.at[ibuf.at[g]], bufs[b], gsems[b])

        def wout_desc(g, b):
            return pltpu.make_async_copy(
                bufs[b], g_hbm.at[pl.ds(row0 + g * WIN, WIN)], wsems[b])

        for b in range(NBUF):
            gather_desc(b, b).start()

        @pl.loop(0, GWIN_PER_W, step=NBUF)
        def _(j):
            for b in range(NBUF):
                g = j + b
                gather_desc(g, b).wait()
                wout_desc(g, b).start()
                b2 = (b + LEAD) % NBUF
                g2 = g + LEAD
                gp = g - (NBUF - LEAD)

                @pl.when(jnp.logical_and(gp >= 0, g2 < GWIN_PER_W))
                def _():
                    wout_desc(gp, b2).wait()
                    gather_desc(g2, b2).start()

        for b in range(NBUF):
            wout_desc(GWIN_PER_W - NBUF + b, b).wait()

    return gather_kernel(x, idx_chunk.reshape(NW, GWIN_PER_W, WIN))


def _edge_mlp_body(xs_ref, xd_ref, ea_ref, w1s_ref, w1d_ref, w1e_ref,
                   b1_ref, w2_ref, b2_ref, m_ref):
    bf = jnp.bfloat16
    h = (
        jnp.dot(xs_ref[...].astype(bf), w1s_ref[...], preferred_element_type=jnp.float32)
        + jnp.dot(xd_ref[...].astype(bf), w1d_ref[...], preferred_element_type=jnp.float32)
        + jnp.dot(ea_ref[...], w1e_ref[...], preferred_element_type=jnp.float32)
        + b1_ref[...]
    )
    h = _mish(h)
    h = jnp.dot(h.astype(bf), w2_ref[...],
                preferred_element_type=jnp.float32) + b2_ref[...]
    m_ref[...] = _mish(h)


def _edge_mlp(g, ea_chunk, W1s, W1d, W1e, b1, W2, b2, e_ch):
    nblk = e_ch // EBLK
    return pl.pallas_call(
        _edge_mlp_body,
        grid=(nblk,),
        in_specs=[
            pl.BlockSpec((EBLK, D), lambda i: (i, 0)),
            pl.BlockSpec((EBLK, D), lambda i, _n=nblk: (i + _n, 0)),
            pl.BlockSpec((EBLK, ED), lambda i: (i, 0)),
            pl.BlockSpec((D, D), lambda i: (0, 0)),
            pl.BlockSpec((D, D), lambda i: (0, 0)),
            pl.BlockSpec((ED, D), lambda i: (0, 0)),
            pl.BlockSpec((1, D), lambda i: (0, 0)),
            pl.BlockSpec((D, D), lambda i: (0, 0)),
            pl.BlockSpec((1, D), lambda i: (0, 0)),
        ],
        out_specs=pl.BlockSpec((EBLK, D), lambda i: (i, 0)),
        out_shape=jax.ShapeDtypeStruct((e_ch, D), jnp.float32),
    )(g, g, ea_chunk, W1s, W1d, W1e, b1, W2, b2)


def _sc_scatter_add(m, dst_chunk, e_ch):
    mesh = plsc.VectorSubcoreMesh(core_axis_name="c", subcore_axis_name="s")
    SWIN_PER_W = e_ch // (NW * WIN)

    @functools.partial(
        pl.kernel,
        out_type=jax.ShapeDtypeStruct((NC, N_AGG, D), jnp.float32),
        mesh=mesh,
        scratch_types=[
            pltpu.VMEM((SWIN_PER_W, WIN), jnp.int32),
            [pltpu.VMEM((WIN, D), jnp.float32) for _ in range(2)],
            [pltpu.SemaphoreType.DMA for _ in range(2)],
            pltpu.VMEM_SHARED((N_AGG, D), jnp.float32),
        ],
    )
    def scatter_kernel(m_hbm, di_hbm, out_hbm, idx_v, mbufs, msems, agg_sh):
        cid = lax.axis_index("c")
        sid = lax.axis_index("s")
        wid = sid * NC + cid
        base = wid * SWIN_PER_W * WIN

        def mread_desc(s, u):
            return pltpu.make_async_copy(
                m_hbm.at[pl.ds(base + s * WIN, WIN)], mbufs[u], msems[u])

        @pl.loop(0, WIN)
        def _(r):
            @pl.loop(0, D, step=16)
            def _(q):
                mbufs[0][r, pl.ds(q, 16)] = jnp.zeros((16,), jnp.float32)

        zrows = (WIN, WIN, WIN, WIN, ROWS_PER_SUB - 4 * WIN)
        off = sid * ROWS_PER_SUB
        for i, zr in enumerate(zrows):
            pltpu.sync_copy(
                mbufs[0].at[pl.ds(0, zr)],
                agg_sh.at[pl.ds(off + i * WIN, zr)],
            )

        pltpu.sync_copy(di_hbm.at[wid], idx_v)
        plsc.subcore_barrier()

        mread_desc(0, 0).start()
        mread_desc(1, 1).start()

        @pl.loop(0, SWIN_PER_W, step=2)
        def _(j):
            for u in range(2):
                s = j + u
                mread_desc(s, u).wait()
                pltpu.sync_copy(mbufs[u], agg_sh.at[idx_v.at[s]], add=True)

                @pl.when(s + 2 < SWIN_PER_W)
                def _():
                    mread_desc(s + 2, u).start()

        plsc.subcore_barrier()
        pltpu.sync_copy(
            agg_sh.at[pl.ds(sid * ROWS_PER_SUB, ROWS_PER_SUB)],
            out_hbm.at[cid].at[pl.ds(sid * ROWS_PER_SUB, ROWS_PER_SUB)],
        )

    return scatter_kernel(m, dst_chunk.reshape(NW, SWIN_PER_W, WIN))


def _node_mlp_body(x_ref, *refs):
    p_refs = refs[:K_CH]
    u1x_ref, u1a_ref, c1_ref, u2_ref, c2_ref, g_ref, bt_ref, o_ref = refs[K_CH:]
    x = x_ref[...]
    agg = sum(p[0] + p[1] for p in p_refs)
    u = (
        jnp.dot(x, u1x_ref[...], preferred_element_type=jnp.float32)
        + jnp.dot(agg, u1a_ref[...], preferred_element_type=jnp.float32)
        + c1_ref[...]
    )
    o = jnp.dot(_mish(u), u2_ref[...], preferred_element_type=jnp.float32) + c2_ref[...]
    r = x + o
    mu = jnp.mean(r, axis=-1, keepdims=True)
    var = jnp.mean((r - mu) ** 2, axis=-1, keepdims=True)
    o_ref[...] = (r - mu) * jax.lax.rsqrt(var + 1e-5) * g_ref[...] + bt_ref[...]


def _node_mlp(x, partials, U1, c1, U2, c2, gamma, beta):
    grid = (N_NODES // NBLK,)
    pspec = pl.BlockSpec((NC, NBLK, D), lambda i: (0, i, 0))
    wspec = pl.BlockSpec((D, D), lambda i: (0, 0))
    vspec = pl.BlockSpec((1, D), lambda i: (0, 0))
    return pl.pallas_call(
        _node_mlp_body,
        grid=grid,
        in_specs=[pl.BlockSpec((NBLK, D), lambda i: (i, 0))]
                 + [pspec] * K_CH
                 + [wspec, wspec, vspec, wspec, vspec, vspec, vspec],
        out_specs=pl.BlockSpec((NBLK, D), lambda i: (i, 0)),
        out_shape=jax.ShapeDtypeStruct((N_NODES, D), jnp.float32),
    )(x, *partials, U1[:D], U1[D:], c1.reshape(1, D), U2, c2.reshape(1, D),
      gamma.reshape(1, D), beta.reshape(1, D))


def kernel(x, edge_index, edge_attr, W1, b1, W2, b2, U1, c1, U2, c2, gamma, beta):
    pad = E_PAD - N_EDGES
    src = jnp.concatenate([edge_index[0], jnp.zeros((pad,), jnp.int32)])
    dst = jnp.concatenate([edge_index[1], jnp.full((pad,), N_NODES, jnp.int32)])
    ea = jnp.concatenate([edge_attr, jnp.zeros((pad, ED), jnp.float32)], axis=0)

    bf = jnp.bfloat16
    W1s, W1d = W1[:D].astype(bf), W1[D:2 * D].astype(bf)
    W1e, b1r = W1[2 * D:], b1.reshape(1, D)
    W2b, b2r = W2.astype(bf), b2.reshape(1, D)

    partials = []
    base = 0
    for e_ch in CH_SIZES:
        sl = slice(base, base + e_ch)
        base += e_ch
        idx_k = jnp.concatenate([src[sl], dst[sl]])
        g_k = _sc_gather(x, idx_k, e_ch)
        m_k = _edge_mlp(g_k, ea[sl], W1s, W1d, W1e, b1r, W2b, b2r, e_ch)
        partials.append(_sc_scatter_add(m_k, dst[sl], e_ch))

    return _node_mlp(x, partials, U1, c1, U2, c2, gamma, beta)
